# baseline (speedup 1.0000x reference)
import functools

import jax
import jax.numpy as jnp
from jax import lax
from jax.experimental import pallas as pl
from jax.experimental.pallas import tpu as pltpu

_H = 256
_NROWS = 160000
_NSEG = 10000
_R = 640
_W = 128
_SB = 2000

_DN = (((1,), (1,)), ((), ()))


def _phi_reduce_body(idx_ref, x_ref, w1_ref, b1_ref, w2_ref, b2_ref, acc_ref):
    i = pl.program_id(0)

    @pl.when(i == 0)
    def _init():
        acc_ref[...] = jnp.zeros_like(acc_ref)

    x = x_ref[...]
    h = jnp.maximum(
        lax.dot_general(x, w1_ref[...], _DN, preferred_element_type=jnp.float32)
        + b1_ref[...], 0.0)
    h = jnp.maximum(
        lax.dot_general(h.astype(jnp.bfloat16), w2_ref[...], _DN,
                        preferred_element_type=jnp.float32)
        + b2_ref[...], 0.0)
    h_aug = jnp.concatenate(
        [h.astype(jnp.bfloat16), jnp.ones((_R, 128), jnp.bfloat16)], axis=1)

    idx = idx_ref[0]
    s0 = jnp.min(idx)
    max_idx = jnp.max(idx)
    s1 = jnp.minimum((s0 // 8) * 8, _NSEG - _W) + _W
    n_pass = 1 + jnp.maximum(0, (max_idx - s1 + _W) // _W)

    def pass_body(_, s):
        base = jnp.minimum((s // 8) * 8, _NSEG - _W)
        lanes = lax.broadcasted_iota(jnp.int32, (_W, _R), 0) + base
        idx_b = jnp.broadcast_to(idx, (_W, _R))
        oh = ((idx_b == lanes) & (idx_b >= s)).astype(jnp.bfloat16)
        contrib = jnp.dot(oh, h_aug, preferred_element_type=jnp.float32)
        acc_ref[pl.ds(base, _W), :] += contrib
        return base + _W

    lax.fori_loop(0, n_pass, pass_body, s0)


def _rho_body(acc_ref, w3_ref, b3_ref, w4_ref, b4_ref, out_ref):
    a = acc_ref[...]
    pooled = a[:, :_H] / jnp.maximum(a[:, _H:_H + 1], 1.0)
    o = jnp.maximum(
        lax.dot_general(pooled.astype(jnp.bfloat16), w3_ref[...], _DN,
                        preferred_element_type=jnp.float32) + b3_ref[...], 0.0)
    o = jnp.maximum(
        lax.dot_general(o.astype(jnp.bfloat16), w4_ref[...], _DN,
                        preferred_element_type=jnp.float32) + b4_ref[...], 0.0)
    out_ref[...] = o


@jax.jit
def kernel(x, molecule_idx, W1, b1, W2, b2, W3, b3, W4, b4):
    nchunks = _NROWS // _R
    idx3 = molecule_idx.astype(jnp.int32).reshape(nchunks, 1, _R)
    bf = jnp.bfloat16

    acc = pl.pallas_call(
        _phi_reduce_body,
        grid=(nchunks,),
        in_specs=[
            pl.BlockSpec((1, 1, _R), lambda i: (i, 0, 0)),
            pl.BlockSpec((_R, _H), lambda i: (i, 0)),
            pl.BlockSpec((_H, _H), lambda i: (0, 0)),
            pl.BlockSpec((1, _H), lambda i: (0, 0)),
            pl.BlockSpec((_H, _H), lambda i: (0, 0)),
            pl.BlockSpec((1, _H), lambda i: (0, 0)),
        ],
        out_specs=pl.BlockSpec((_NSEG, _H + 128), lambda i: (0, 0)),
        out_shape=jax.ShapeDtypeStruct((_NSEG, _H + 128), jnp.float32),
    )(idx3, x.astype(bf), W1.astype(bf), b1.reshape(1, _H),
      W2.astype(bf), b2.reshape(1, _H))

    out = pl.pallas_call(
        _rho_body,
        grid=(_NSEG // _SB,),
        in_specs=[
            pl.BlockSpec((_SB, _H + 128), lambda i: (i, 0)),
            pl.BlockSpec((_H, _H), lambda i: (0, 0)),
            pl.BlockSpec((1, _H), lambda i: (0, 0)),
            pl.BlockSpec((_H, _H), lambda i: (0, 0)),
            pl.BlockSpec((1, _H), lambda i: (0, 0)),
        ],
        out_specs=pl.BlockSpec((_SB, _H), lambda i: (i, 0)),
        out_shape=jax.ShapeDtypeStruct((_NSEG, _H), jnp.float32),
    )(acc, W3.astype(bf), b3.reshape(1, _H), W4.astype(bf), b4.reshape(1, _H))
    return out


# speedup vs baseline: 2.3919x; 2.3919x over previous
import functools

import jax
import jax.numpy as jnp
from jax import lax
from jax.experimental import pallas as pl
from jax.experimental.pallas import tpu as pltpu

_H = 256
_NROWS = 160000
_NSEG = 10000
_R = 3200
_W = 256
_SB = 2000

_DN = (((1,), (1,)), ((), ()))


def _phi_reduce_body(idx_ref, x_ref, w1_ref, b1_ref, w2_ref, b2_ref,
                     acc_ref, cnt_ref):
    i = pl.program_id(0)

    @pl.when(i == 0)
    def _init():
        acc_ref[...] = jnp.zeros_like(acc_ref)
        cnt_ref[...] = jnp.zeros_like(cnt_ref)

    x = x_ref[...].astype(jnp.bfloat16)
    h = jnp.maximum(
        lax.dot_general(x, w1_ref[...], _DN, preferred_element_type=jnp.float32)
        + b1_ref[...], 0.0)
    h = jnp.maximum(
        lax.dot_general(h.astype(jnp.bfloat16), w2_ref[...], _DN,
                        preferred_element_type=jnp.float32)
        + b2_ref[...], 0.0)
    hb = h.astype(jnp.bfloat16)

    idx = idx_ref[0]
    s0 = jnp.min(idx)
    max_idx = jnp.max(idx)
    s1 = jnp.minimum((s0 // 8) * 8, _NSEG - _W) + _W
    n_pass = 1 + jnp.maximum(0, (max_idx - s1 + _W) // _W)

    def pass_body(_, s):
        base = jnp.minimum((s // 8) * 8, _NSEG - _W)
        lanes = lax.broadcasted_iota(jnp.int32, (_W, _R), 0) + base
        idx_b = jnp.broadcast_to(idx, (_W, _R))
        sel = (idx_b == lanes) & (idx_b >= s)
        oh = sel.astype(jnp.bfloat16)
        contrib = jnp.dot(oh, hb, preferred_element_type=jnp.float32)
        acc_ref[pl.ds(base, _W), :] += contrib
        cnt_ref[pl.ds(base, _W), :] += jnp.sum(
            sel.astype(jnp.float32), axis=1, keepdims=True)
        return base + _W

    lax.fori_loop(0, n_pass, pass_body, s0)


def _rho_body(acc_ref, cnt_ref, w3_ref, b3_ref, w4_ref, b4_ref, out_ref):
    pooled = acc_ref[...] / jnp.maximum(cnt_ref[...], 1.0)
    o = jnp.maximum(
        lax.dot_general(pooled.astype(jnp.bfloat16), w3_ref[...], _DN,
                        preferred_element_type=jnp.float32) + b3_ref[...], 0.0)
    o = jnp.maximum(
        lax.dot_general(o.astype(jnp.bfloat16), w4_ref[...], _DN,
                        preferred_element_type=jnp.float32) + b4_ref[...], 0.0)
    out_ref[...] = o


@jax.jit
def kernel(x, molecule_idx, W1, b1, W2, b2, W3, b3, W4, b4):
    nchunks = _NROWS // _R
    idx3 = molecule_idx.astype(jnp.int32).reshape(nchunks, 1, _R)
    bf = jnp.bfloat16

    acc, cnt = pl.pallas_call(
        _phi_reduce_body,
        grid=(nchunks,),
        in_specs=[
            pl.BlockSpec((1, 1, _R), lambda i: (i, 0, 0)),
            pl.BlockSpec((_R, _H), lambda i: (i, 0)),
            pl.BlockSpec((_H, _H), lambda i: (0, 0)),
            pl.BlockSpec((1, _H), lambda i: (0, 0)),
            pl.BlockSpec((_H, _H), lambda i: (0, 0)),
            pl.BlockSpec((1, _H), lambda i: (0, 0)),
        ],
        out_specs=[
            pl.BlockSpec((_NSEG, _H), lambda i: (0, 0)),
            pl.BlockSpec((_NSEG, 1), lambda i: (0, 0)),
        ],
        out_shape=[
            jax.ShapeDtypeStruct((_NSEG, _H), jnp.float32),
            jax.ShapeDtypeStruct((_NSEG, 1), jnp.float32),
        ],
    )(idx3, x, W1.astype(bf), b1.reshape(1, _H),
      W2.astype(bf), b2.reshape(1, _H))

    out = pl.pallas_call(
        _rho_body,
        grid=(_NSEG // _SB,),
        in_specs=[
            pl.BlockSpec((_SB, _H), lambda i: (i, 0)),
            pl.BlockSpec((_SB, 1), lambda i: (i, 0)),
            pl.BlockSpec((_H, _H), lambda i: (0, 0)),
            pl.BlockSpec((1, _H), lambda i: (0, 0)),
            pl.BlockSpec((_H, _H), lambda i: (0, 0)),
            pl.BlockSpec((1, _H), lambda i: (0, 0)),
        ],
        out_specs=pl.BlockSpec((_SB, _H), lambda i: (i, 0)),
        out_shape=jax.ShapeDtypeStruct((_NSEG, _H), jnp.float32),
    )(acc, cnt, W3.astype(bf), b3.reshape(1, _H), W4.astype(bf),
      b4.reshape(1, _H))
    return out
